# baseline (speedup 1.0000x reference)
import functools

import jax
import jax.numpy as jnp
from jax import lax
from jax.experimental import pallas as pl
from jax.experimental.pallas import tpu as pltpu
from jax.experimental.pallas import tpu_sc as plsc

N_SIDE = 3
CUBE = N_SIDE ** 3
CH = 16
CHUNK = 128
NWORKERS = 32
LANES = 16
H1, H2 = 128, 64
BN = 2048


def _sc_gather(vol, i0, i1, i2, npad, pg):
  nchunks = npad // (NWORKERS * CHUNK)
  pts_per_w = npad // NWORKERS
  mesh = plsc.VectorSubcoreMesh(core_axis_name="c", subcore_axis_name="s")

  deltas = []
  for o in range(CUBE):
    dx, dy, dz = o // 9 - 1, (o // 3) % 3 - 1, o % 3 - 1
    deltas.append(dx * pg * pg + dy * pg + dz)

  @functools.partial(
      pl.kernel,
      mesh=mesh,
      compiler_params=pltpu.CompilerParams(use_tc_tiling_on_sc=False),
      out_type=jax.ShapeDtypeStruct((npad // CHUNK, CHUNK, CUBE, CH),
                                    jnp.float32),
      scratch_types=[
          pltpu.VMEM((CHUNK,), jnp.int32),
          pltpu.VMEM((CHUNK,), jnp.int32),
          pltpu.VMEM((CHUNK,), jnp.int32),
          pltpu.VMEM((CUBE * CHUNK,), jnp.int32),
          pltpu.VMEM((CHUNK, CUBE, CH), jnp.float32),
          pltpu.SemaphoreType.DMA,
          pltpu.SemaphoreType.DMA,
      ],
  )
  def k(vol_hbm, i0_hbm, i1_hbm, i2_hbm, x_hbm, c0, c1, c2, idxb, xstage,
        gsem, osem):
    wid = lax.axis_index("s") * 2 + lax.axis_index("c")

    def body(c, carry):
      pbase = wid * pts_per_w + c * CHUNK
      cg = wid * nchunks + c
      pltpu.sync_copy(i0_hbm.at[pl.ds(pbase, CHUNK)], c0)
      pltpu.sync_copy(i1_hbm.at[pl.ds(pbase, CHUNK)], c1)
      pltpu.sync_copy(i2_hbm.at[pl.ds(pbase, CHUNK)], c2)
      base3s = []
      for v in range(CHUNK // LANES):
        a0 = c0[pl.ds(v * LANES, LANES)]
        a1 = c1[pl.ds(v * LANES, LANES)]
        a2 = c2[pl.ds(v * LANES, LANES)]
        base3s.append(((a0 + 1) * pg + (a1 + 1)) * pg + (a2 + 1))
      for o in range(CUBE):
        for v in range(CHUNK // LANES):
          idxb[pl.ds(o * CHUNK + v * LANES, LANES)] = base3s[v] + deltas[o]
      gcps = [
          pltpu.async_copy(
              vol_hbm.at[idxb.at[pl.ds(o * CHUNK, CHUNK)]],
              xstage.at[:, o, :], gsem)
          for o in range(CUBE)
      ]
      for cp in gcps:
        cp.wait()
      pltpu.async_copy(xstage, x_hbm.at[cg], osem).wait()
      return carry

    lax.fori_loop(0, nchunks, body, 0)

  return k(vol, i0, i1, i2)


def _mlp_body(x_ref, w1_ref, b1_ref, w2_ref, b2_ref, w3_ref, b3_ref, o_ref):
  h = jnp.dot(x_ref[...], w1_ref[...], preferred_element_type=jnp.float32)
  h = jnp.maximum(h + b1_ref[...], 0.0)
  h = jnp.dot(h, w2_ref[...], preferred_element_type=jnp.float32)
  h = jnp.maximum(h + b2_ref[...], 0.0)
  t = jnp.dot(h, w3_ref[...], preferred_element_type=jnp.float32)
  o_ref[...] = jnp.tanh(t + b3_ref[...])


def _mlp(x, w1p, b1p, w2, b2, w3, b3, npad):
  in_dim = CUBE * CH
  return pl.pallas_call(
      _mlp_body,
      grid=(npad // BN,),
      in_specs=[
          pl.BlockSpec((BN, in_dim), lambda i: (i, 0)),
          pl.BlockSpec((in_dim, H1), lambda i: (0, 0)),
          pl.BlockSpec((1, H1), lambda i: (0, 0)),
          pl.BlockSpec((H1, H2), lambda i: (0, 0)),
          pl.BlockSpec((1, H2), lambda i: (0, 0)),
          pl.BlockSpec((H2, 1), lambda i: (0, 0)),
          pl.BlockSpec((1, 1), lambda i: (0, 0)),
      ],
      out_specs=pl.BlockSpec((BN, 1), lambda i: (i, 0)),
      out_shape=jax.ShapeDtypeStruct((npad, 1), jnp.float32),
  )(x, w1p, b1p, w2, b2, w3, b3)


def kernel(tsdf, weights, features, indices, W1, b1, W2, b2, W3, b3):
  g = tsdf.shape[0]
  pg = g + 2
  feat = features.shape[-1]
  n = indices.shape[0]
  step = NWORKERS * CHUNK
  npad = -(-n // step) * step

  packed = jnp.concatenate(
      [tsdf[..., None] + 0.1, weights[..., None], features], axis=-1)
  vol = jnp.pad(packed, ((1, 1), (1, 1), (1, 1), (0, CH - 2 - feat)))
  vol = vol.reshape(pg * pg * pg, CH)

  idx = jnp.pad(indices.astype(jnp.int32), ((0, npad - n), (0, 0)))
  i0, i1, i2 = idx[:, 0], idx[:, 1], idx[:, 2]

  deltas = []
  for o in range(CUBE):
    dx, dy, dz = o // 9 - 1, (o // 3) % 3 - 1, o % 3 - 1
    deltas.append(dx * pg * pg + dy * pg + dz)
  base3 = ((i0 + 1) * pg + (i1 + 1)) * pg + (i2 + 1)
  flat = base3[:, None] + jnp.array(deltas, jnp.int32)[None, :]
  x = jnp.take(vol, flat.reshape(-1), axis=0)
  x = x.reshape(npad, CUBE * CH)

  w1r = W1.reshape(CUBE, 2 + feat, H1)
  w1p = jnp.pad(w1r, ((0, 0), (0, CH - 2 - feat), (0, 0)))
  w1p = w1p.reshape(CUBE * CH, H1)
  b1p = (b1 - 0.1 * jnp.sum(w1r[:, 0, :], axis=0)).reshape(1, H1)

  out = _mlp(x, w1p, b1p, W2, b2.reshape(1, H2), W3, b3.reshape(1, 1), npad)
  return out[:n]
